# baseline (speedup 1.0000x reference)
import functools

import numpy as np

import jax
import jax.numpy as jnp
from jax import lax
from jax.experimental import pallas as pl
from jax.experimental.pallas import tpu as pltpu
import jax.experimental.pallas.tpu_sc as plsc

N = 10000
E = 160000
HID = 256
D_OUT = 128
EPS = 1e-7

NPAD = 10240
TRASH = NPAD - 1
B = 128
EP_TILE = NPAD
KCH = EP_TILE // B
E_PAD = 16 * EP_TILE
RB = 400
NB = N // RB

NQ = 2560
NQ2 = 2688
TR2 = NQ2 - 1
B2 = 128
KB2 = 24
CAP4 = 16 * KB2 * B2
RB4 = 160
NB4 = NQ // RB4

_mesh = plsc.VectorSubcoreMesh(core_axis_name="c", subcore_axis_name="s")


@functools.partial(
    pl.kernel,
    out_type=jax.ShapeDtypeStruct((4, 2, NQ2, 128), jnp.float32),
    mesh=_mesh,
    scratch_types=[
        pltpu.VMEM((KB2, B2), jnp.int32),
        pltpu.VMEM((B2,), jnp.int32),
        pltpu.VMEM((B2,), jnp.int32),
        pltpu.VMEM((B2,), jnp.int32),
        pltpu.VMEM((B2,), jnp.int32),
        pltpu.VMEM((B2, 128), jnp.float32),
        pltpu.VMEM((B2, 128), jnp.float32),
        pltpu.VMEM((B2, 128), jnp.float32),
        pltpu.VMEM((B2, 128), jnp.float32),
        pltpu.VMEM_SHARED((NQ2, 128), jnp.float32),
        pltpu.VMEM_SHARED((NQ2, 128), jnp.float32),
        pltpu.SemaphoreType.DMA,
        pltpu.SemaphoreType.DMA,
    ],
)
def _sc_agg(m_hbm, packed_hbm, zeros_hbm, out_hbm,
            packed_v, sidx0_v, didx0_v, sidx1_v, didx1_v,
            rows0_v, rows1_v, flo_v, fhi_v, alo_sh, ahi_sh, sem0, sem1):
    cc = lax.axis_index("c")
    s = lax.axis_index("s")
    nr = NQ2 // 16

    def unpack(j, sidx, didx):
        for k in range(B2 // 16):
            p = packed_v[j, pl.ds(k * 16, 16)]
            sidx[pl.ds(k * 16, 16)] = p & 0xFFFF
            didx[pl.ds(k * 16, 16)] = lax.shift_right_logical(p, 16)

    def gather(sidx, buf, sem):
        return pltpu.make_async_copy(m_hbm.at[sidx], buf, sem)

    def expand(ibuf):
        @plsc.parallel_loop(0, B2, unroll=4)
        def _(r):
            for k in range(8):
                w = lax.bitcast_convert_type(ibuf[r, pl.ds(k * 16, 16)],
                                             jnp.int32)
                lo = lax.bitcast_convert_type(
                    lax.shift_left(w, 16), jnp.float32)
                hi = lax.bitcast_convert_type(
                    w & jnp.int32(-65536), jnp.float32)
                fr = flo_v if k < 4 else fhi_v
                kk = k % 4
                fr[r, pl.ds(kk * 32, 16)] = lo
                fr[r, pl.ds(kk * 32 + 16, 16)] = hi

    for p in range(2):
        qid = cc * 2 + p
        pltpu.sync_copy(packed_hbm.at[qid, pl.ds(s * KB2, KB2)], packed_v)
        pltpu.sync_copy(zeros_hbm, alo_sh.at[pl.ds(s * nr, nr)])
        pltpu.sync_copy(zeros_hbm, ahi_sh.at[pl.ds(s * nr, nr)])
        plsc.subcore_barrier()

        unpack(0, sidx0_v, didx0_v)
        unpack(1, sidx1_v, didx1_v)
        gather(sidx0_v, rows0_v, sem0).start()
        gather(sidx1_v, rows1_v, sem1).start()

        def pair(i, carry):
            j2 = 2 * i + 2
            gather(sidx0_v, rows0_v, sem0).wait()
            expand(rows0_v)
            pltpu.sync_copy(flo_v, alo_sh.at[didx0_v], add=True)
            pltpu.sync_copy(fhi_v, ahi_sh.at[didx0_v], add=True)

            @pl.when(i + 1 < KB2 // 2)
            def _():
                unpack(j2, sidx0_v, didx0_v)
                gather(sidx0_v, rows0_v, sem0).start()

            gather(sidx1_v, rows1_v, sem1).wait()
            expand(rows1_v)
            pltpu.sync_copy(flo_v, alo_sh.at[didx1_v], add=True)
            pltpu.sync_copy(fhi_v, ahi_sh.at[didx1_v], add=True)

            @pl.when(i + 1 < KB2 // 2)
            def _():
                unpack(j2 + 1, sidx1_v, didx1_v)
                gather(sidx1_v, rows1_v, sem1).start()

            return carry

        lax.fori_loop(0, KB2 // 2, pair, 0)
        plsc.subcore_barrier()
        pltpu.sync_copy(alo_sh.at[pl.ds(s * nr, nr)],
                        out_hbm.at[qid, 0, pl.ds(s * nr, nr)])
        pltpu.sync_copy(ahi_sh.at[pl.ds(s * nr, nr)],
                        out_hbm.at[qid, 1, pl.ds(s * nr, nr)])
        plsc.subcore_barrier()


@functools.partial(
    pl.kernel,
    out_type=jax.ShapeDtypeStruct((NPAD, 128), jnp.float32),
    mesh=_mesh,
    scratch_types=[
        pltpu.VMEM((KCH, B), jnp.int32),
        pltpu.VMEM((B, 128), jnp.float32),
        pltpu.VMEM_SHARED((NPAD, 128), jnp.float32),
        pltpu.SemaphoreType.DMA,
    ],
)
def _sc_deg(dst2_hbm, ones_hbm, zeros_hbm, out_hbm,
            dst_v, ones_v, deg_sh, sem):
    cc = lax.axis_index("c")
    s = lax.axis_index("s")
    pltpu.sync_copy(dst2_hbm.at[pl.ds(s * KCH, KCH)], dst_v)
    pltpu.sync_copy(ones_hbm, ones_v)
    pltpu.sync_copy(zeros_hbm, deg_sh.at[pl.ds(s * (NPAD // 16), NPAD // 16)])
    plsc.subcore_barrier()

    def chunk(j, carry):
        pltpu.sync_copy(ones_v, deg_sh.at[dst_v.at[j]], add=True)
        return carry

    lax.fori_loop(0, KCH, chunk, 0)
    plsc.subcore_barrier()
    nr = NPAD // 16

    @pl.when(cc == 0)
    def _():
        pltpu.sync_copy(deg_sh.at[pl.ds(s * nr, nr)],
                        out_hbm.at[pl.ds(s * nr, nr)])


def _row_norm(v):
    n = jnp.sqrt(jnp.sum(v * v, axis=-1, keepdims=True))
    return jnp.maximum(n, EPS)


def _exp_map(v, sc):
    n = _row_norm(v)
    return jnp.tanh(sc * n) * v / (sc * n)


def _log_map(y, sc):
    n = _row_norm(y)
    scn = jnp.clip(sc * n, EPS, 1.0 - 1e-5)
    atan = 0.5 * jnp.log((1.0 + scn) / (1.0 - scn))
    return atan * y / (sc * n)


def _pack_bf16_pairs(m):
    be = lax.bitcast_convert_type(m[:, :128], jnp.int32) + 0x8000
    bo = lax.bitcast_convert_type(m[:, 128:], jnp.int32) + 0x8000
    packed = lax.shift_right_logical(be, 16) | (bo & jnp.int32(-65536))
    return lax.bitcast_convert_type(packed, jnp.float32)


def _tc_enc_body(x_ref, we_ref, be_ref, w_ref, b_ref, c_ref, out_ref):
    sc = jnp.sqrt(c_ref[0, 0])
    t = jnp.dot(x_ref[...], we_ref[...],
                preferred_element_type=jnp.float32) + be_ref[...]
    ht = _log_map(_exp_map(t, sc), sc)
    m = jnp.dot(ht, w_ref[...], preferred_element_type=jnp.float32) + b_ref[...]
    out_ref[0] = _pack_bf16_pairs(m)


def _tc_enc(x, w_enc, b_enc, w0, b0, c2d):
    return pl.pallas_call(
        _tc_enc_body,
        grid=(NB,),
        in_specs=[
            pl.BlockSpec((RB, HID), lambda i: (i, 0)),
            pl.BlockSpec((HID, HID), lambda i: (0, 0)),
            pl.BlockSpec((1, HID), lambda i: (0, 0)),
            pl.BlockSpec((HID, HID), lambda i: (0, 0)),
            pl.BlockSpec((1, HID), lambda i: (0, 0)),
            pl.BlockSpec((1, 1), lambda i: (0, 0)),
        ],
        out_specs=pl.BlockSpec((1, RB, 128),
                               lambda i: ((i * RB) // (NPAD // 2),
                                          (i * RB % (NPAD // 2)) // RB, 0)),
        out_shape=jax.ShapeDtypeStruct((2, NPAD // 2, 128), jnp.float32),
    )(x, w_enc, b_enc, w0, b0, c2d)


def _make_tc_mid_body(head):
    def body(agg_ref, deg_ref, w_ref, b_ref, c_ref, out_ref):
        sc = jnp.sqrt(c_ref[0, 0])
        a = jnp.concatenate([agg_ref[0, 0], agg_ref[0, 1]], axis=1)
        d = jnp.maximum(deg_ref[:, 0:1], 1.0)
        a = a / d
        h = _exp_map(a, sc)
        h = _exp_map(_log_map(h, sc), sc)
        ht = _log_map(h, sc)
        m = jnp.dot(ht, w_ref[...],
                    preferred_element_type=jnp.float32) + b_ref[...]
        if head:
            out_ref[0] = m
        else:
            out_ref[0] = _pack_bf16_pairs(m)
    return body


def _tc_mid(agg, deg, w, b, c2d, head):
    wout = w.shape[1]
    if head:
        out_specs = pl.BlockSpec((1, RB4, 128),
                                 lambda q, i: (0, q * NB4 + i, 0))
        out_shape = jax.ShapeDtypeStruct((1, NPAD, 128), jnp.float32)
    else:
        out_specs = pl.BlockSpec(
            (1, RB4, 128),
            lambda q, i: ((q * NB4 + i) // 32, (q * NB4 + i) % 32, 0))
        out_shape = jax.ShapeDtypeStruct((2, NPAD // 2, 128), jnp.float32)
    return pl.pallas_call(
        _make_tc_mid_body(head),
        grid=(4, NB4),
        in_specs=[
            pl.BlockSpec((1, 2, RB4, 128), lambda q, i: (q, 0, i, 0)),
            pl.BlockSpec((RB4, 128), lambda q, i: (q * NB4 + i, 0)),
            pl.BlockSpec((HID, wout), lambda q, i: (0, 0)),
            pl.BlockSpec((1, wout), lambda q, i: (0, 0)),
            pl.BlockSpec((1, 1), lambda q, i: (0, 0)),
        ],
        out_specs=out_specs,
        out_shape=out_shape,
    )(agg, deg, w, b, c2d)


_PE = np.array([32 * (j // 16) + j % 16 for j in range(128)])
_PERM256 = np.concatenate([_PE, _PE + 16])


@jax.jit
def kernel(x, edge_index, c_param, W_enc, b_enc, W0, b0, W1, b1, W2, b2,
           W_head, b_head):
    c2d = (jnp.abs(c_param) + 1e-5).reshape(1, 1).astype(jnp.float32)
    ei = edge_index.astype(jnp.int32)
    src = ei[0]
    dst = ei[1]
    npad = E_PAD - E
    dst_pad = jnp.concatenate([dst, jnp.full((npad,), TRASH, jnp.int32)])
    dst2 = dst_pad.reshape(E_PAD // B, B)
    zeros128 = jnp.zeros((NPAD // 16, 128), jnp.float32)
    zerosq = jnp.zeros((NQ2 // 16, 128), jnp.float32)
    ones128 = jnp.ones((B, 128), jnp.float32)

    deg = _sc_deg(dst2, ones128, zeros128)

    q = dst // NQ
    onehot = (q[None, :] == jnp.arange(4, dtype=jnp.int32)[:, None])
    ranks = jnp.cumsum(onehot.astype(jnp.int32), axis=1)
    rank = jnp.take_along_axis(ranks, q[None, :], axis=0)[0]
    pos = q * CAP4 + rank - 1
    pk = src | ((dst - q * NQ) << 16)
    part = jnp.full((4 * CAP4,), TR2 << 16, jnp.int32).at[pos].set(pk)
    packed = part.reshape(4, CAP4 // B2, B2)

    w0p = W0[:, _PERM256]
    b0p = b0[_PERM256].reshape(1, -1)
    w1p = W1[:, _PERM256]
    b1p = b1[_PERM256].reshape(1, -1)
    w2p = W2[:, _PERM256]
    b2p = b2[_PERM256].reshape(1, -1)

    m = _tc_enc(x, W_enc, b_enc.reshape(1, -1), w0p, b0p, c2d)
    agg = _sc_agg(m.reshape(NPAD, 128), packed, zerosq)
    m = _tc_mid(agg, deg, w1p, b1p, c2d, head=False)
    agg = _sc_agg(m.reshape(NPAD, 128), packed, zerosq)
    m = _tc_mid(agg, deg, w2p, b2p, c2d, head=False)
    agg = _sc_agg(m.reshape(NPAD, 128), packed, zerosq)
    out = _tc_mid(agg, deg, W_head, b_head.reshape(1, -1), c2d, head=True)
    return out[0, :N]


# speedup vs baseline: 4.0971x; 4.0971x over previous
import functools

import jax
import jax.numpy as jnp
from jax import lax
from jax.experimental import pallas as pl
from jax.experimental.pallas import tpu as pltpu
import jax.experimental.pallas.tpu_sc as plsc

N = 10000
E = 160000
HID = 256
D_OUT = 128
EPS = 1e-7

NPAD = 10240
TRASH = NPAD - 1
B = 128
EP_TILE = NPAD
KCH = EP_TILE // B
E_PAD = 16 * EP_TILE
RB = 400
NB = N // RB

_mesh = plsc.VectorSubcoreMesh(core_axis_name="c", subcore_axis_name="s")


@functools.partial(
    pl.kernel,
    out_type=jax.ShapeDtypeStruct((2, NPAD, 128), jnp.float32),
    mesh=_mesh,
    scratch_types=[
        pltpu.VMEM((KCH, B), jnp.int32),
        pltpu.VMEM((B,), jnp.int32),
        pltpu.VMEM((B,), jnp.int32),
        pltpu.VMEM((B,), jnp.int32),
        pltpu.VMEM((B,), jnp.int32),
        pltpu.VMEM((B, 128), jnp.float32),
        pltpu.VMEM((B, 128), jnp.float32),
        pltpu.VMEM_SHARED((NPAD, 128), jnp.float32),
        pltpu.SemaphoreType.DMA,
        pltpu.SemaphoreType.DMA,
    ],
)
def _sc_agg(m_hbm, packed_hbm, zeros_hbm, out_hbm,
            packed_v, sidx0_v, didx0_v, sidx1_v, didx1_v,
            rows0_v, rows1_v, agg_sh, sem0, sem1):
    cc = lax.axis_index("c")
    s = lax.axis_index("s")
    pltpu.sync_copy(packed_hbm.at[pl.ds(s * KCH, KCH)], packed_v)
    pltpu.sync_copy(zeros_hbm, agg_sh.at[pl.ds(s * (NPAD // 16), NPAD // 16)])
    plsc.subcore_barrier()

    row_off = cc * N

    def unpack(j, sidx, didx):
        for k in range(B // 16):
            p = packed_v[j, pl.ds(k * 16, 16)]
            sidx[pl.ds(k * 16, 16)] = (p & 0xFFFF) + row_off
            didx[pl.ds(k * 16, 16)] = lax.shift_right_logical(p, 16)

    def gather(sidx, buf, sem):
        return pltpu.make_async_copy(m_hbm.at[sidx], buf, sem)

    unpack(0, sidx0_v, didx0_v)
    gather(sidx0_v, rows0_v, sem0).start()

    def pair(i, carry):
        j1 = 2 * i + 1
        unpack(j1, sidx1_v, didx1_v)
        gather(sidx0_v, rows0_v, sem0).wait()
        gather(sidx1_v, rows1_v, sem1).start()
        pltpu.sync_copy(rows0_v, agg_sh.at[didx0_v], add=True)

        @pl.when(i + 1 < KCH // 2)
        def _():
            unpack(j1 + 1, sidx0_v, didx0_v)

        gather(sidx1_v, rows1_v, sem1).wait()

        @pl.when(i + 1 < KCH // 2)
        def _():
            gather(sidx0_v, rows0_v, sem0).start()

        pltpu.sync_copy(rows1_v, agg_sh.at[didx1_v], add=True)
        return carry

    lax.fori_loop(0, KCH // 2, pair, 0)
    plsc.subcore_barrier()
    nr = NPAD // 16
    pltpu.sync_copy(agg_sh.at[pl.ds(s * nr, nr)],
                    out_hbm.at[cc, pl.ds(s * nr, nr)])


@functools.partial(
    pl.kernel,
    out_type=jax.ShapeDtypeStruct((NPAD, 128), jnp.float32),
    mesh=_mesh,
    scratch_types=[
        pltpu.VMEM((KCH, B), jnp.int32),
        pltpu.VMEM((B, 128), jnp.float32),
        pltpu.VMEM_SHARED((NPAD, 128), jnp.float32),
        pltpu.SemaphoreType.DMA,
    ],
)
def _sc_deg(dst2_hbm, ones_hbm, zeros_hbm, out_hbm,
            dst_v, ones_v, deg_sh, sem):
    cc = lax.axis_index("c")
    s = lax.axis_index("s")
    pltpu.sync_copy(dst2_hbm.at[pl.ds(s * KCH, KCH)], dst_v)
    pltpu.sync_copy(ones_hbm, ones_v)
    pltpu.sync_copy(zeros_hbm, deg_sh.at[pl.ds(s * (NPAD // 16), NPAD // 16)])
    plsc.subcore_barrier()

    def chunk(j, carry):
        pltpu.sync_copy(ones_v, deg_sh.at[dst_v.at[j]], add=True)
        return carry

    lax.fori_loop(0, KCH, chunk, 0)
    plsc.subcore_barrier()
    nr = NPAD // 16

    @pl.when(cc == 0)
    def _():
        pltpu.sync_copy(deg_sh.at[pl.ds(s * nr, nr)],
                        out_hbm.at[pl.ds(s * nr, nr)])


def _row_norm(v):
    n = jnp.sqrt(jnp.sum(v * v, axis=-1, keepdims=True))
    return jnp.maximum(n, EPS)


def _exp_map(v, sc):
    n = _row_norm(v)
    return jnp.tanh(sc * n) * v / (sc * n)


def _log_map(y, sc):
    n = _row_norm(y)
    scn = jnp.clip(sc * n, EPS, 1.0 - 1e-5)
    atan = 0.5 * jnp.log((1.0 + scn) / (1.0 - scn))
    return atan * y / (sc * n)


def _tc_enc_body(x_ref, we_ref, be_ref, w_ref, b_ref, c_ref, out_ref):
    sc = jnp.sqrt(c_ref[0, 0])
    t = jnp.dot(x_ref[...], we_ref[...],
                preferred_element_type=jnp.float32) + be_ref[...]
    ht = _log_map(_exp_map(t, sc), sc)
    m = jnp.dot(ht, w_ref[...], preferred_element_type=jnp.float32) + b_ref[...]
    out_ref[0] = m[:, :128]
    out_ref[1] = m[:, 128:]


def _tc_enc(x, w_enc, b_enc, w0, b0, c2d):
    return pl.pallas_call(
        _tc_enc_body,
        grid=(NB,),
        in_specs=[
            pl.BlockSpec((RB, HID), lambda i: (i, 0)),
            pl.BlockSpec((HID, HID), lambda i: (0, 0)),
            pl.BlockSpec((1, HID), lambda i: (0, 0)),
            pl.BlockSpec((HID, HID), lambda i: (0, 0)),
            pl.BlockSpec((1, HID), lambda i: (0, 0)),
            pl.BlockSpec((1, 1), lambda i: (0, 0)),
        ],
        out_specs=pl.BlockSpec((2, RB, 128), lambda i: (0, i, 0)),
        out_shape=jax.ShapeDtypeStruct((2, N, 128), jnp.float32),
    )(x, w_enc, b_enc, w0, b0, c2d)


def _make_tc_mid_body(nh):
    def body(agg_ref, deg_ref, w_ref, b_ref, c_ref, out_ref):
        sc = jnp.sqrt(c_ref[0, 0])
        a = jnp.concatenate([agg_ref[0], agg_ref[1]], axis=1)
        d = jnp.maximum(deg_ref[:, 0:1], 1.0)
        a = a / d
        h = _exp_map(a, sc)
        h = _exp_map(_log_map(h, sc), sc)
        ht = _log_map(h, sc)
        m = jnp.dot(ht, w_ref[...],
                    preferred_element_type=jnp.float32) + b_ref[...]
        for k in range(nh):
            out_ref[k] = m[:, k * 128:(k + 1) * 128]
    return body


def _tc_mid(agg, deg, w, b, c2d, nh):
    return pl.pallas_call(
        _make_tc_mid_body(nh),
        grid=(NB,),
        in_specs=[
            pl.BlockSpec((2, RB, 128), lambda i: (0, i, 0)),
            pl.BlockSpec((RB, 128), lambda i: (i, 0)),
            pl.BlockSpec((HID, nh * 128), lambda i: (0, 0)),
            pl.BlockSpec((1, nh * 128), lambda i: (0, 0)),
            pl.BlockSpec((1, 1), lambda i: (0, 0)),
        ],
        out_specs=pl.BlockSpec((nh, RB, 128), lambda i: (0, i, 0)),
        out_shape=jax.ShapeDtypeStruct((nh, N, 128), jnp.float32),
    )(agg, deg, w, b, c2d)


@jax.jit
def kernel(x, edge_index, c_param, W_enc, b_enc, W0, b0, W1, b1, W2, b2,
           W_head, b_head):
    c2d = (jnp.abs(c_param) + 1e-5).reshape(1, 1).astype(jnp.float32)
    ei = edge_index.astype(jnp.int32)
    src = ei[0]
    dst = ei[1]
    npad = E_PAD - E
    src_pad = jnp.concatenate([src, jnp.zeros((npad,), jnp.int32)])
    dst_pad = jnp.concatenate([dst, jnp.full((npad,), TRASH, jnp.int32)])
    packed = (src_pad | (dst_pad << 16)).reshape(E_PAD // B, B)
    dst2 = dst_pad.reshape(E_PAD // B, B)
    zeros128 = jnp.zeros((NPAD // 16, 128), jnp.float32)
    ones128 = jnp.ones((B, 128), jnp.float32)

    deg = _sc_deg(dst2, ones128, zeros128)

    m = _tc_enc(x, W_enc, b_enc.reshape(1, -1), W0, b0.reshape(1, -1), c2d)
    agg = _sc_agg(m.reshape(2 * N, 128), packed, zeros128)
    m = _tc_mid(agg, deg, W1, b1.reshape(1, -1), c2d, nh=2)
    agg = _sc_agg(m.reshape(2 * N, 128), packed, zeros128)
    m = _tc_mid(agg, deg, W2, b2.reshape(1, -1), c2d, nh=2)
    agg = _sc_agg(m.reshape(2 * N, 128), packed, zeros128)
    out = _tc_mid(agg, deg, W_head, b_head.reshape(1, -1), c2d, nh=1)
    return out[0]
